# baseline (speedup 1.0000x reference)
import functools

import jax
import jax.numpy as jnp
from jax.experimental import pallas as pl
from jax.experimental.pallas import tpu as pltpu

_NB = 16


def _fused_qnet_kernel(x4_ref, c1w_ref, c1b_ref, c2w_ref, c2b_ref,
                       c3w_ref, c3b_ref, w1_ref, b1_ref, w2_ref, b2_ref,
                       o_ref, p1_ref, h1_ref, p2_ref, p3_ref, flat_ref):
    nb = x4_ref.shape[0]
    x4 = x4_ref[...]


    def blk(sr, sc):
        k = (sr % 2) * 2 + (sc % 2)
        r0, c0 = sr // 2, sc // 2
        return x4[:, k, r0:r0 + 10, c0:c0 + 10, :]

    for par in range(4):
        ph, pw = divmod(par, 2)
        for t in range(4):
            ti, tj = divmod(t, 2)
            p1_ref[:, t * 64:(t + 1) * 64] = (
                blk(ph + ti, pw + tj).reshape(nb * 100, 64))
        a1 = jnp.dot(p1_ref[...], c1w_ref[...],
                     preferred_element_type=jnp.float32)
        a1 = jnp.maximum(a1 + c1b_ref[...], 0.0)
        h1_ref[:, par * 128:(par + 1) * 128] = a1.astype(jnp.bfloat16)
    h1 = h1_ref[...].reshape(nb, 10, 10, 512)

    for i in range(4):
        ai, pi = divmod(i, 2)
        for j in range(4):
            aj, pj = divmod(j, 2)
            lo = (pi * 2 + pj) * 128
            p2_ref[:, (i * 4 + j) * 128:(i * 4 + j + 1) * 128] = (
                h1[:, ai:ai + 9, aj:aj + 9, lo:lo + 128].reshape(nb * 81, 128))
    a2 = jnp.dot(p2_ref[...], c2w_ref[...],
                 preferred_element_type=jnp.float32)
    a2 = jnp.maximum(a2 + c2b_ref[...], 0.0)
    h2 = a2.astype(jnp.bfloat16).reshape(nb, 9, 9, 128)

    for t in range(9):
        i, j = divmod(t, 3)
        p3_ref[:, t * 128:(t + 1) * 128] = (
            h2[:, i:i + 7, j:j + 7, :].reshape(nb * 49, 128))
    a3 = jnp.dot(p3_ref[...], c3w_ref[...],
                 preferred_element_type=jnp.float32)
    a3 = jnp.maximum(a3 + c3b_ref[...], 0.0)
    h3v = a3.astype(jnp.bfloat16)[:, :64].reshape(nb, 49, 64)
    for p in range(49):
        flat_ref[:, p * 64:(p + 1) * 64] = h3v[:, p, :]
    h3 = flat_ref[...]

    hid = jnp.dot(h3, w1_ref[...], preferred_element_type=jnp.float32)
    hid = jnp.maximum(hid + b1_ref[...], 0.0).astype(jnp.bfloat16)
    q = jnp.dot(hid, w2_ref[...], preferred_element_type=jnp.float32)
    o_ref[...] = q + b2_ref[...]


def _resident(shape):
    nd = len(shape)
    return pl.BlockSpec(shape, lambda i: (0,) * nd)


@jax.jit
def kernel(c1w, c1b, c2w, c2b, c3w, c3b, w1, b1, w2, b2, x):
    B = x.shape[0]
    nb = _NB

    xs = jnp.transpose(x, (0, 2, 3, 1)).astype(jnp.bfloat16)
    xs = xs.reshape(B, 21, 4, 21, 4, 4).transpose(0, 1, 3, 2, 4, 5)
    xs = xs.reshape(B, 21, 21, 64)
    xs = jnp.pad(xs, ((0, 0), (0, 1), (0, 1), (0, 0)))
    x4 = xs.reshape(B, 11, 2, 11, 2, 64).transpose(0, 2, 4, 1, 3, 5)
    x4 = x4.reshape(B, 4, 11, 11, 64)

    c1w_s = c1w.reshape(2, 4, 2, 4, 4, c1w.shape[1])
    c1w_s = c1w_s.transpose(0, 2, 1, 3, 4, 5).reshape(256, c1w.shape[1])

    c2w_p = jnp.pad(c2w.reshape(16, 32, 128),
                    ((0, 0), (0, 96), (0, 0))).reshape(2048, 128)
    c3w_p = jnp.pad(c3w.reshape(9, 64, 128),
                    ((0, 0), (0, 64), (0, 0))).reshape(1152, 128)

    out = pl.pallas_call(
        _fused_qnet_kernel,
        out_shape=jax.ShapeDtypeStruct((B, 128), jnp.float32),
        grid=(B // nb,),
        in_specs=[
            pl.BlockSpec((nb, 4, 11, 11, 64), lambda i: (i, 0, 0, 0, 0)),
            _resident(c1w_s.shape),
            _resident(c1b.shape),
            _resident(c2w_p.shape),
            _resident(c2b.shape),
            _resident(c3w_p.shape),
            _resident(c3b.shape),
            _resident(w1.shape),
            _resident(b1.shape),
            _resident(w2.shape),
            _resident(b2.shape),
        ],
        out_specs=pl.BlockSpec((nb, 128), lambda i: (i, 0)),
        scratch_shapes=[
            pltpu.VMEM((nb * 100, 256), jnp.bfloat16),
            pltpu.VMEM((nb * 100, 512), jnp.bfloat16),
            pltpu.VMEM((nb * 81, 2048), jnp.bfloat16),
            pltpu.VMEM((nb * 49, 1152), jnp.bfloat16),
            pltpu.VMEM((nb, 3136), jnp.bfloat16),
        ],
        compiler_params=pltpu.CompilerParams(
            dimension_semantics=("parallel",)),
    )(x4, c1w_s, c1b, c2w_p, c2b, c3w_p, c3b, w1, b1, w2, b2)
    return out[:, :4]


# speedup vs baseline: 1.3122x; 1.3122x over previous
import functools

import jax
import jax.numpy as jnp
from jax.experimental import pallas as pl
from jax.experimental.pallas import tpu as pltpu

_NB = 32


def _fused_qnet_kernel(x4_ref, c1w_ref, c1b_ref, c2w_ref, c2b_ref,
                       c3w_ref, c3b_ref, w1_ref, b1_ref, w2_ref, b2_ref,
                       o_ref, h1_ref, flat_ref):
    nb = x4_ref.shape[0]
    x4 = x4_ref[...]

    def blk(sr, sc):
        k = (sr % 2) * 2 + (sc % 2)
        r0, c0 = sr // 2, sc // 2
        return x4[:, k, r0:r0 + 10, c0:c0 + 10, :]

    for par in range(4):
        ph, pw = divmod(par, 2)
        p1 = jnp.concatenate(
            [blk(ph + ti, pw + tj) for ti in range(2) for tj in range(2)],
            axis=-1).reshape(nb * 100, 256)
        a1 = jnp.dot(p1, c1w_ref[...],
                     preferred_element_type=jnp.float32)
        a1 = jnp.maximum(a1 + c1b_ref[...], 0.0)
        h1_ref[:, par * 128:(par + 1) * 128] = a1.astype(jnp.bfloat16)
    h1 = h1_ref[...].reshape(nb, 10, 10, 512)

    cols2 = []
    for i in range(4):
        ai, pi = divmod(i, 2)
        for j in range(4):
            aj, pj = divmod(j, 2)
            lo = (pi * 2 + pj) * 128
            cols2.append(h1[:, ai:ai + 9, aj:aj + 9, lo:lo + 128])
    p2 = jnp.concatenate(cols2, axis=-1).reshape(nb * 81, 2048)
    a2 = jnp.dot(p2, c2w_ref[...], preferred_element_type=jnp.float32)
    a2 = jnp.maximum(a2 + c2b_ref[...], 0.0)
    h2 = a2.astype(jnp.bfloat16).reshape(nb, 9, 9, 128)

    p3 = jnp.concatenate(
        [h2[:, i:i + 7, j:j + 7, :] for i in range(3) for j in range(3)],
        axis=-1).reshape(nb * 49, 1152)
    a3 = jnp.dot(p3, c3w_ref[...], preferred_element_type=jnp.float32)
    a3 = jnp.maximum(a3 + c3b_ref[...], 0.0)
    h3v = a3.astype(jnp.bfloat16)[:, :64].reshape(nb, 49, 64)
    for p in range(49):
        flat_ref[:, p * 64:(p + 1) * 64] = h3v[:, p, :]
    h3 = flat_ref[...]

    hid = jnp.dot(h3, w1_ref[...], preferred_element_type=jnp.float32)
    hid = jnp.maximum(hid + b1_ref[...], 0.0).astype(jnp.bfloat16)
    q = jnp.dot(hid, w2_ref[...], preferred_element_type=jnp.float32)
    o_ref[...] = q + b2_ref[...]


def _resident(shape):
    nd = len(shape)
    return pl.BlockSpec(shape, lambda i: (0,) * nd)


@jax.jit
def kernel(c1w, c1b, c2w, c2b, c3w, c3b, w1, b1, w2, b2, x):
    B = x.shape[0]
    nb = _NB

    xs = jnp.transpose(x, (0, 2, 3, 1)).astype(jnp.bfloat16)
    xs = xs.reshape(B, 21, 4, 21, 4, 4).transpose(0, 1, 3, 2, 4, 5)
    xs = xs.reshape(B, 21, 21, 64)
    xs = jnp.pad(xs, ((0, 0), (0, 1), (0, 1), (0, 0)))
    x4 = xs.reshape(B, 11, 2, 11, 2, 64).transpose(0, 2, 4, 1, 3, 5)
    x4 = x4.reshape(B, 4, 11, 11, 64)

    c1w_s = c1w.reshape(2, 4, 2, 4, 4, c1w.shape[1])
    c1w_s = c1w_s.transpose(0, 2, 1, 3, 4, 5).reshape(256, c1w.shape[1])

    c2w_p = jnp.pad(c2w.reshape(16, 32, 128),
                    ((0, 0), (0, 96), (0, 0))).reshape(2048, 128)
    c3w_p = jnp.pad(c3w.reshape(9, 64, 128),
                    ((0, 0), (0, 64), (0, 0))).reshape(1152, 128)

    out = pl.pallas_call(
        _fused_qnet_kernel,
        out_shape=jax.ShapeDtypeStruct((B, 128), jnp.float32),
        grid=(B // nb,),
        in_specs=[
            pl.BlockSpec((nb, 4, 11, 11, 64), lambda i: (i, 0, 0, 0, 0)),
            _resident(c1w_s.shape),
            _resident(c1b.shape),
            _resident(c2w_p.shape),
            _resident(c2b.shape),
            _resident(c3w_p.shape),
            _resident(c3b.shape),
            _resident(w1.shape),
            _resident(b1.shape),
            _resident(w2.shape),
            _resident(b2.shape),
        ],
        out_specs=pl.BlockSpec((nb, 128), lambda i: (i, 0)),
        scratch_shapes=[
            pltpu.VMEM((nb * 100, 512), jnp.bfloat16),
            pltpu.VMEM((nb, 3136), jnp.bfloat16),
        ],
        compiler_params=pltpu.CompilerParams(
            dimension_semantics=("parallel",)),
    )(x4, c1w_s, c1b, c2w_p, c2b, c3w_p, c3b, w1, b1, w2, b2)
    return out[:, :4]
